# Initial kernel scaffold; baseline (speedup 1.0000x reference)
import jax
import jax.numpy as jnp
from jax.experimental import pallas as pl


def kernel(x, edge_index, batch, emb, W1, b1, g1, be1, W2, b2, g2, be2, L1W, L1b, L2W, L2b, L3W, L3b):
    raise NotImplementedError("write your pallas kernel here")


# speedup vs baseline: 12.8242x; 12.8242x over previous
import functools

import jax
import jax.numpy as jnp
from jax import lax
from jax.experimental import pallas as pl
from jax.experimental.pallas import tpu as pltpu
from jax.experimental.pallas import tpu_sc as plsc

N = 10000
E = 160000
F = 80
NC = 2
NS = 16
NW = NC * NS

XCHUNKS = 98
XC_PER_TILE = XCHUNKS * 128
XPAD = NW * XC_PER_TILE

ECHUNKS = 40
E_PER_TILE = ECHUNKS * 128
EPAD = NW * E_PER_TILE

RROWS = N + 16
DROWS = N + 240

_mesh = plsc.VectorSubcoreMesh(core_axis_name="c", subcore_axis_name="s")
_sc_params = pltpu.CompilerParams(use_tc_tiling_on_sc=False,
                                  needs_layout_passes=False)


@functools.partial(
    pl.kernel,
    out_type=[
        jax.ShapeDtypeStruct((NW, XC_PER_TILE * 2), jnp.float32),
        jax.ShapeDtypeStruct((NC, NS, 640, 16), jnp.float32),
    ],
    mesh=_mesh,
    scratch_types=[
        pltpu.VMEM((XC_PER_TILE,), jnp.int32),
        pltpu.VMEM((2 * N,), jnp.float32),
        pltpu.VMEM((XC_PER_TILE * 2,), jnp.float32),
        pltpu.VMEM((ECHUNKS, 128), jnp.int32),
        pltpu.VMEM((128, 16), jnp.float32),
        pltpu.VMEM((640, 16), jnp.float32),
        pltpu.VMEM_SHARED((DROWS, 16), jnp.float32),
    ],
    compiler_params=_sc_params,
)
def _embdeg(x_hbm, emb_hbm, dst_hbm, ones_hbm, zer_hbm,
            h0_hbm, degp_hbm, xidx, embv, outb, didx, onesb, bounce, deg_sp):
    c = lax.axis_index("c")
    s = lax.axis_index("s")
    w = s * NC + c

    pltpu.sync_copy(x_hbm.at[w], xidx)
    pltpu.sync_copy(emb_hbm, embv)
    pltpu.sync_copy(dst_hbm.at[w], didx)
    pltpu.sync_copy(ones_hbm, onesb)
    pltpu.sync_copy(zer_hbm, bounce)
    pltpu.sync_copy(bounce, deg_sp.at[pl.ds(s * 640, 640)])
    plsc.subcore_barrier()

    @pl.loop(0, ECHUNKS)
    def _deg(j):
        pltpu.sync_copy(onesb, deg_sp.at[didx.at[j]], add=True)

    lanes = lax.iota(jnp.int32, 16)

    @pl.loop(0, XC_PER_TILE // 16)
    def _emb(i):
        idx16 = xidx[pl.ds(i * 16, 16)]
        fa = idx16 * 2
        a = plsc.load_gather(embv, [fa])
        b = plsc.load_gather(embv, [fa + 1])
        pos = i * 32 + lanes * 2
        plsc.store_scatter(outb, [pos], a)
        plsc.store_scatter(outb, [pos + 1], b)

    pltpu.sync_copy(outb, h0_hbm.at[w])
    plsc.subcore_barrier()
    pltpu.sync_copy(deg_sp.at[pl.ds(s * 640, 640)], bounce)
    pltpu.sync_copy(bounce, degp_hbm.at[c].at[s])


@functools.partial(
    pl.kernel,
    out_type=jax.ShapeDtypeStruct((NC, NS, 625, F), jnp.float32),
    mesh=_mesh,
    scratch_types=[
        pltpu.VMEM((ECHUNKS, 128), jnp.int32),
        pltpu.VMEM((ECHUNKS, 128), jnp.int32),
        pltpu.VMEM((128, F), jnp.float32),
        pltpu.VMEM((125, F), jnp.float32),
        pltpu.VMEM_SHARED((RROWS, F), jnp.float32),
    ],
    compiler_params=_sc_params,
)
def _scatter(p_hbm, src_hbm, dst_hbm, zer_hbm, r_hbm,
             sidx, didx, msg, bounce, r_sp):
    c = lax.axis_index("c")
    s = lax.axis_index("s")
    w = s * NC + c

    pltpu.sync_copy(src_hbm.at[w], sidx)
    pltpu.sync_copy(dst_hbm.at[w], didx)
    pltpu.sync_copy(zer_hbm, bounce)
    for k in range(5):
        pltpu.sync_copy(bounce, r_sp.at[pl.ds(s * 625 + k * 125, 125)])
    plsc.subcore_barrier()

    @pl.loop(0, ECHUNKS)
    def _edges(j):
        pltpu.sync_copy(p_hbm.at[sidx.at[j]], msg)
        pltpu.sync_copy(msg, r_sp.at[didx.at[j]], add=True)

    plsc.subcore_barrier()
    for k in range(5):
        pltpu.sync_copy(r_sp.at[pl.ds(s * 625 + k * 125, 125)], bounce)
        pltpu.sync_copy(bounce, r_hbm.at[c].at[s].at[pl.ds(k * 125, 125)])


def _t0_body(dp_ref, h0_ref, p_ref, dinv_ref):
    dp = dp_ref[...]
    deg = lax.slice(dp, (0, 0), (N, 1)) + lax.slice(dp, (0, 1), (N, 2)) + 1.0
    dinv = 1.0 / jnp.sqrt(deg)
    dinv_ref[...] = dinv
    p_ref[...] = h0_ref[...] * dinv


_t0 = pl.pallas_call(
    _t0_body,
    out_shape=[
        jax.ShapeDtypeStruct((N, F), jnp.float32),
        jax.ShapeDtypeStruct((N, 1), jnp.float32),
    ],
)


def _gcn_body(r0_ref, r1_ref, p_ref, dinv_ref, w_ref, b_ref, g_ref, be_ref,
              h_ref, pn_ref):
    dinv = dinv_ref[...]
    sm = (r0_ref[...] + r1_ref[...] + p_ref[...]) * dinv
    y = jnp.dot(sm, w_ref[...], preferred_element_type=jnp.float32) + b_ref[...]
    mu = jnp.mean(y, axis=0, keepdims=True)
    d = y - mu
    var = jnp.mean(d * d, axis=0, keepdims=True)
    h = jnp.maximum(d / jnp.sqrt(var + 1e-5) * g_ref[...] + be_ref[...], 0.0)
    h_ref[...] = h
    pn_ref[...] = h * dinv


_gcn = pl.pallas_call(
    _gcn_body,
    out_shape=[
        jax.ShapeDtypeStruct((N, F), jnp.float32),
        jax.ShapeDtypeStruct((N, F), jnp.float32),
    ],
)


def _head_body(z_ref, w1e, w1o, b1e, b1o, w2e, w2o, b2e, b2o, w3, b3, out_ref):
    z = z_ref[...]
    a = jnp.dot(z, w1e[...], preferred_element_type=jnp.float32) + b1e[...]
    b = jnp.dot(z, w1o[...], preferred_element_type=jnp.float32) + b1o[...]
    a = jnp.maximum(jnp.maximum(a, b), 0.0)
    u = jnp.dot(a, w2e[...], preferred_element_type=jnp.float32) + b2e[...]
    v = jnp.dot(a, w2o[...], preferred_element_type=jnp.float32) + b2o[...]
    u = jnp.maximum(jnp.maximum(u, v), 0.0)
    out_ref[...] = jnp.dot(u, w3[...], preferred_element_type=jnp.float32) + b3[...]


_head = pl.pallas_call(
    _head_body,
    out_shape=jax.ShapeDtypeStruct((250, 20), jnp.float32),
)


def kernel(x, edge_index, batch, emb, W1, b1, g1, be1, W2, b2, g2, be2,
           L1W, L1b, L2W, L2b, L3W, L3b):
    x_flat = x.reshape(-1)
    x_pad = jnp.concatenate(
        [x_flat, jnp.arange(XPAD - N * 40, dtype=jnp.int32) % N]
    ).reshape(NW, XC_PER_TILE)
    src = edge_index[0]
    dst = edge_index[1]
    pad_n = EPAD - E
    src_p = jnp.concatenate(
        [src, jnp.arange(pad_n, dtype=jnp.int32) % N]).reshape(NW, ECHUNKS, 128)
    dst_p = jnp.concatenate(
        [dst, N + (jnp.arange(pad_n, dtype=jnp.int32) % 16)]
    ).reshape(NW, ECHUNKS, 128)
    ones_upd = jnp.ones((128, 16), jnp.float32)
    zer_deg = jnp.zeros((640, 16), jnp.float32)
    zer_row = jnp.zeros((125, F), jnp.float32)

    h0_pairs, degp = _embdeg(x_pad, emb.reshape(-1), dst_p, ones_upd, zer_deg)
    h0 = h0_pairs.reshape(-1)[: N * F].reshape(N, F)
    deg_cols = degp[..., 0].reshape(NC, NS * 640)[:, :N].T

    p1, dinv = _t0(deg_cols, h0)

    rp1 = _scatter(p1, src_p, dst_p, zer_row).reshape(NC, N, F)
    h1, p2 = _gcn(rp1[0], rp1[1], p1, dinv, W1, b1.reshape(1, F),
                  g1.reshape(1, F), be1.reshape(1, F))

    rp2 = _scatter(p2, src_p, dst_p, zer_row).reshape(NC, N, F)
    h2, _ = _gcn(rp2[0], rp2[1], p2, dinv, W2, b2.reshape(1, F),
                 g2.reshape(1, F), be2.reshape(1, F))

    z = h2.reshape(250, 40 * F)
    out = _head(z,
                L1W[:, 0::2], L1W[:, 1::2],
                L1b[0::2].reshape(1, 400), L1b[1::2].reshape(1, 400),
                L2W[:, 0::2], L2W[:, 1::2],
                L2b[0::2].reshape(1, 50), L2b[1::2].reshape(1, 50),
                L3W, L3b.reshape(1, 20))
    return out


# speedup vs baseline: 14.3735x; 1.1208x over previous
import functools

import jax
import jax.numpy as jnp
from jax import lax
from jax.experimental import pallas as pl
from jax.experimental.pallas import tpu as pltpu
from jax.experimental.pallas import tpu_sc as plsc

N = 10000
E = 160000
F = 80
NC = 2
NS = 16
NW = NC * NS

XCHUNKS = 98
XC_PER_TILE = XCHUNKS * 128
XPAD = NW * XC_PER_TILE

ECHUNKS = 40
E_PER_TILE = ECHUNKS * 128
EPAD = NW * E_PER_TILE

RROWS = N + 16
DROWS = N + 240

_mesh = plsc.VectorSubcoreMesh(core_axis_name="c", subcore_axis_name="s")
_sc_params = pltpu.CompilerParams(use_tc_tiling_on_sc=False,
                                  needs_layout_passes=False)


@functools.partial(
    pl.kernel,
    out_type=[
        jax.ShapeDtypeStruct((NW, XC_PER_TILE * 2), jnp.float32),
        jax.ShapeDtypeStruct((NC, NS, 640, 16), jnp.float32),
    ],
    mesh=_mesh,
    scratch_types=[
        pltpu.VMEM((XC_PER_TILE,), jnp.int32),
        pltpu.VMEM((2 * N,), jnp.float32),
        pltpu.VMEM((XC_PER_TILE * 2,), jnp.float32),
        pltpu.VMEM((ECHUNKS, 128), jnp.int32),
        pltpu.VMEM((128, 16), jnp.float32),
        pltpu.VMEM((640, 16), jnp.float32),
        pltpu.VMEM_SHARED((DROWS, 16), jnp.float32),
        pltpu.SemaphoreType.DMA,
    ],
    compiler_params=_sc_params,
)
def _embdeg(x_hbm, emb_hbm, dst_hbm, ones_hbm, zer_hbm,
            h0_hbm, degp_hbm, xidx, embv, outb, didx, onesb, bounce, deg_sp,
            deg_sem):
    c = lax.axis_index("c")
    s = lax.axis_index("s")
    w = s * NC + c

    pltpu.sync_copy(x_hbm.at[w], xidx)
    pltpu.sync_copy(emb_hbm, embv)
    pltpu.sync_copy(dst_hbm.at[w], didx)
    pltpu.sync_copy(ones_hbm, onesb)
    pltpu.sync_copy(zer_hbm, bounce)
    pltpu.sync_copy(bounce, deg_sp.at[pl.ds(s * 640, 640)])
    plsc.subcore_barrier()

    @pl.loop(0, ECHUNKS // 4)
    def _deg(jj):
        j = jj * 4
        for t in range(4):
            pltpu.async_copy(onesb, deg_sp.at[didx.at[j + t]], deg_sem, add=True)
        for t in range(4):
            pltpu.make_async_copy(onesb, deg_sp.at[didx.at[j + t]], deg_sem).wait()

    lanes = lax.iota(jnp.int32, 16)

    @pl.loop(0, XC_PER_TILE // 16, unroll=4)
    def _emb(i):
        idx16 = xidx[pl.ds(i * 16, 16)]
        fa = idx16 * 2
        a = plsc.load_gather(embv, [fa])
        b = plsc.load_gather(embv, [fa + 1])
        pos = i * 32 + lanes * 2
        plsc.store_scatter(outb, [pos], a)
        plsc.store_scatter(outb, [pos + 1], b)

    pltpu.sync_copy(outb, h0_hbm.at[w])
    plsc.subcore_barrier()
    pltpu.sync_copy(deg_sp.at[pl.ds(s * 640, 640)], bounce)
    pltpu.sync_copy(bounce, degp_hbm.at[c].at[s])


@functools.partial(
    pl.kernel,
    out_type=jax.ShapeDtypeStruct((NC, NS, 625, F), jnp.float32),
    mesh=_mesh,
    scratch_types=[
        pltpu.VMEM((ECHUNKS, 128), jnp.int32),
        pltpu.VMEM((ECHUNKS, 128), jnp.int32),
        pltpu.VMEM((128, F), jnp.float32),
        pltpu.VMEM((128, F), jnp.float32),
        pltpu.VMEM((125, F), jnp.float32),
        pltpu.VMEM_SHARED((RROWS, F), jnp.float32),
        pltpu.SemaphoreType.DMA,
        pltpu.SemaphoreType.DMA,
    ],
    compiler_params=_sc_params,
)
def _scatter(p_hbm, src_hbm, dst_hbm, zer_hbm, r_hbm,
             sidx, didx, msg0, msg1, bounce, r_sp, sem0, sem1):
    c = lax.axis_index("c")
    s = lax.axis_index("s")
    w = s * NC + c

    pltpu.sync_copy(src_hbm.at[w], sidx)
    pltpu.sync_copy(dst_hbm.at[w], didx)
    pltpu.sync_copy(zer_hbm, bounce)
    for k in range(5):
        pltpu.sync_copy(bounce, r_sp.at[pl.ds(s * 625 + k * 125, 125)])
    plsc.subcore_barrier()

    pltpu.async_copy(p_hbm.at[sidx.at[0]], msg0, sem0)

    @pl.loop(0, ECHUNKS // 2)
    def _edges(jj):
        j = jj * 2
        pltpu.async_copy(p_hbm.at[sidx.at[j + 1]], msg1, sem1)
        pltpu.make_async_copy(p_hbm.at[sidx.at[j]], msg0, sem0).wait()
        pltpu.sync_copy(msg0, r_sp.at[didx.at[j]], add=True)

        @pl.when(jj < ECHUNKS // 2 - 1)
        def _next():
            pltpu.async_copy(p_hbm.at[sidx.at[j + 2]], msg0, sem0)

        pltpu.make_async_copy(p_hbm.at[sidx.at[j + 1]], msg1, sem1).wait()
        pltpu.sync_copy(msg1, r_sp.at[didx.at[j + 1]], add=True)

    plsc.subcore_barrier()
    for k in range(5):
        pltpu.sync_copy(r_sp.at[pl.ds(s * 625 + k * 125, 125)], bounce)
        pltpu.sync_copy(bounce, r_hbm.at[c].at[s].at[pl.ds(k * 125, 125)])


def _t0_body(dp_ref, h0_ref, p_ref, dinv_ref):
    dp = dp_ref[...]
    deg = lax.slice(dp, (0, 0), (N, 1)) + lax.slice(dp, (0, 1), (N, 2)) + 1.0
    dinv = 1.0 / jnp.sqrt(deg)
    dinv_ref[...] = dinv
    p_ref[...] = h0_ref[...] * dinv


_t0 = pl.pallas_call(
    _t0_body,
    out_shape=[
        jax.ShapeDtypeStruct((N, F), jnp.float32),
        jax.ShapeDtypeStruct((N, 1), jnp.float32),
    ],
)


def _gcn_body(r0_ref, r1_ref, p_ref, dinv_ref, w_ref, b_ref, g_ref, be_ref,
              h_ref, pn_ref):
    dinv = dinv_ref[...]
    sm = (r0_ref[...] + r1_ref[...] + p_ref[...]) * dinv
    y = jnp.dot(sm, w_ref[...], preferred_element_type=jnp.float32) + b_ref[...]
    mu = jnp.mean(y, axis=0, keepdims=True)
    d = y - mu
    var = jnp.mean(d * d, axis=0, keepdims=True)
    h = jnp.maximum(d / jnp.sqrt(var + 1e-5) * g_ref[...] + be_ref[...], 0.0)
    h_ref[...] = h
    pn_ref[...] = h * dinv


_gcn = pl.pallas_call(
    _gcn_body,
    out_shape=[
        jax.ShapeDtypeStruct((N, F), jnp.float32),
        jax.ShapeDtypeStruct((N, F), jnp.float32),
    ],
)


def _head_body(z_ref, w1e, w1o, b1e, b1o, w2e, w2o, b2e, b2o, w3, b3, out_ref):
    z = z_ref[...]
    a = jnp.dot(z, w1e[...], preferred_element_type=jnp.float32) + b1e[...]
    b = jnp.dot(z, w1o[...], preferred_element_type=jnp.float32) + b1o[...]
    a = jnp.maximum(jnp.maximum(a, b), 0.0)
    u = jnp.dot(a, w2e[...], preferred_element_type=jnp.float32) + b2e[...]
    v = jnp.dot(a, w2o[...], preferred_element_type=jnp.float32) + b2o[...]
    u = jnp.maximum(jnp.maximum(u, v), 0.0)
    out_ref[...] = jnp.dot(u, w3[...], preferred_element_type=jnp.float32) + b3[...]


_head = pl.pallas_call(
    _head_body,
    out_shape=jax.ShapeDtypeStruct((250, 20), jnp.float32),
)


def kernel(x, edge_index, batch, emb, W1, b1, g1, be1, W2, b2, g2, be2,
           L1W, L1b, L2W, L2b, L3W, L3b):
    x_flat = x.reshape(-1)
    x_pad = jnp.concatenate(
        [x_flat, jnp.arange(XPAD - N * 40, dtype=jnp.int32) % N]
    ).reshape(NW, XC_PER_TILE)
    src = edge_index[0]
    dst = edge_index[1]
    pad_n = EPAD - E
    src_p = jnp.concatenate(
        [src, jnp.arange(pad_n, dtype=jnp.int32) % N]).reshape(NW, ECHUNKS, 128)
    dst_p = jnp.concatenate(
        [dst, N + (jnp.arange(pad_n, dtype=jnp.int32) % 16)]
    ).reshape(NW, ECHUNKS, 128)
    ones_upd = jnp.ones((128, 16), jnp.float32)
    zer_deg = jnp.zeros((640, 16), jnp.float32)
    zer_row = jnp.zeros((125, F), jnp.float32)

    h0_pairs, degp = _embdeg(x_pad, emb.reshape(-1), dst_p, ones_upd, zer_deg)
    h0 = h0_pairs.reshape(-1)[: N * F].reshape(N, F)
    deg_cols = degp[..., 0].reshape(NC, NS * 640)[:, :N].T

    p1, dinv = _t0(deg_cols, h0)

    rp1 = _scatter(p1, src_p, dst_p, zer_row).reshape(NC, N, F)
    h1, p2 = _gcn(rp1[0], rp1[1], p1, dinv, W1, b1.reshape(1, F),
                  g1.reshape(1, F), be1.reshape(1, F))

    rp2 = _scatter(p2, src_p, dst_p, zer_row).reshape(NC, N, F)
    h2, _ = _gcn(rp2[0], rp2[1], p2, dinv, W2, b2.reshape(1, F),
                 g2.reshape(1, F), be2.reshape(1, F))

    z = h2.reshape(250, 40 * F)
    out = _head(z,
                L1W[:, 0::2], L1W[:, 1::2],
                L1b[0::2].reshape(1, 400), L1b[1::2].reshape(1, 400),
                L2W[:, 0::2], L2W[:, 1::2],
                L2b[0::2].reshape(1, 50), L2b[1::2].reshape(1, 50),
                L3W, L3b.reshape(1, 20))
    return out


# speedup vs baseline: 16.9833x; 1.1816x over previous
import functools

import jax
import jax.numpy as jnp
from jax import lax
from jax.experimental import pallas as pl
from jax.experimental.pallas import tpu as pltpu
from jax.experimental.pallas import tpu_sc as plsc

N = 10000
E = 160000
F = 80
NC = 2
NS = 16
NW = NC * NS

XPT = 12500
XSTG = 12504
NCH = 1250
CPW = 39
DROWS = N + 240

_mesh = plsc.VectorSubcoreMesh(core_axis_name="c", subcore_axis_name="s")
_sc_params = pltpu.CompilerParams(use_tc_tiling_on_sc=False,
                                  needs_layout_passes=False)


@functools.partial(
    pl.kernel,
    out_type=[
        jax.ShapeDtypeStruct((N * F,), jnp.float32),
        jax.ShapeDtypeStruct((NC, NS, 640, 16), jnp.float32),
    ],
    mesh=_mesh,
    scratch_types=[
        pltpu.VMEM((XSTG,), jnp.int32),
        pltpu.VMEM((2 * N,), jnp.float32),
        pltpu.VMEM((2 * XPT,), jnp.float32),
        pltpu.VMEM((40, 128), jnp.int32),
        pltpu.VMEM((128, 16), jnp.float32),
        pltpu.VMEM((640, 16), jnp.float32),
        pltpu.VMEM_SHARED((DROWS, 16), jnp.float32),
        pltpu.SemaphoreType.DMA,
    ],
    compiler_params=_sc_params,
)
def _embdeg(x_hbm, emb_hbm, dst_hbm, ones_hbm, zer_hbm,
            h0_hbm, degp_hbm, xidx, embv, outb, didx, onesb, bounce, deg_sp,
            deg_sem):
    c = lax.axis_index("c")
    s = lax.axis_index("s")
    w = s * NC + c
    base = 4 * lax.rem(w, 2)

    xoff = pl.multiple_of(w * XPT - base, 8)
    pltpu.sync_copy(x_hbm.at[pl.ds(xoff, XSTG)], xidx)
    pltpu.sync_copy(emb_hbm, embv)
    pltpu.sync_copy(dst_hbm.at[pl.ds(w * CPW, CPW)], didx.at[pl.ds(0, CPW)])

    @pl.when(w < 2)
    def _stage_extra():
        pltpu.sync_copy(dst_hbm.at[pl.ds(1248 + w, 1)], didx.at[pl.ds(39, 1)])

    pltpu.sync_copy(ones_hbm, onesb)
    pltpu.sync_copy(zer_hbm, bounce)
    pltpu.sync_copy(bounce, deg_sp.at[pl.ds(s * 640, 640)])
    plsc.subcore_barrier()

    @pl.loop(0, CPW // 3)
    def _deg(jj):
        j = jj * 3
        for t in range(3):
            pltpu.async_copy(onesb, deg_sp.at[didx.at[j + t]], deg_sem, add=True)
        for t in range(3):
            pltpu.make_async_copy(onesb, deg_sp.at[didx.at[j + t]], deg_sem).wait()

    @pl.when(w < 2)
    def _deg_extra():
        pltpu.sync_copy(onesb, deg_sp.at[didx.at[39]], add=True)

    lanes = lax.iota(jnp.int32, 16)

    @pl.loop(0, XPT // 16, unroll=4)
    def _emb(i):
        idx16 = xidx[pl.ds(base + i * 16, 16)]
        fa = idx16 * 2
        a = plsc.load_gather(embv, [fa])
        b = plsc.load_gather(embv, [fa + 1])
        pos = i * 32 + lanes * 2
        plsc.store_scatter(outb, [pos], a)
        plsc.store_scatter(outb, [pos + 1], b)

    tl = xidx[pl.ds(base + XPT - 16, 16)]
    fa = tl * 2
    a = plsc.load_gather(embv, [fa])
    b = plsc.load_gather(embv, [fa + 1])
    pos = 2 * (XPT - 16) + lanes * 2
    plsc.store_scatter(outb, [pos], a)
    plsc.store_scatter(outb, [pos + 1], b)

    pltpu.sync_copy(outb, h0_hbm.at[pl.ds(pl.multiple_of(w * 2 * XPT, 8),
                                          2 * XPT)])
    plsc.subcore_barrier()
    pltpu.sync_copy(deg_sp.at[pl.ds(s * 640, 640)], bounce)
    pltpu.sync_copy(bounce, degp_hbm.at[c].at[s])


@functools.partial(
    pl.kernel,
    out_type=[
        jax.ShapeDtypeStruct((N, F), jnp.float32),
        jax.ShapeDtypeStruct((N, F), jnp.float32),
    ],
    mesh=_mesh,
    scratch_types=[
        pltpu.VMEM((40, 128), jnp.int32),
        pltpu.VMEM((40, 128), jnp.int32),
        pltpu.VMEM((128, F), jnp.float32),
        pltpu.VMEM((128, F), jnp.float32),
        pltpu.VMEM((125, F), jnp.float32),
        pltpu.VMEM_SHARED((N, F), jnp.float32),
        pltpu.SemaphoreType.DMA,
        pltpu.SemaphoreType.DMA,
    ],
    compiler_params=_sc_params,
)
def _scatter(p_hbm, src_hbm, dst_hbm, zer_hbm, r0_hbm, r1_hbm,
             sidx, didx, msg0, msg1, bounce, r_sp, sem0, sem1):
    c = lax.axis_index("c")
    s = lax.axis_index("s")
    w = s * NC + c

    pltpu.sync_copy(src_hbm.at[pl.ds(w * CPW, CPW)], sidx.at[pl.ds(0, CPW)])
    pltpu.sync_copy(dst_hbm.at[pl.ds(w * CPW, CPW)], didx.at[pl.ds(0, CPW)])

    @pl.when(w < 2)
    def _stage_extra():
        pltpu.sync_copy(src_hbm.at[pl.ds(1248 + w, 1)], sidx.at[pl.ds(39, 1)])
        pltpu.sync_copy(dst_hbm.at[pl.ds(1248 + w, 1)], didx.at[pl.ds(39, 1)])

    pltpu.sync_copy(zer_hbm, bounce)
    for k in range(5):
        pltpu.sync_copy(bounce, r_sp.at[pl.ds(s * 625 + k * 125, 125)])
    plsc.subcore_barrier()

    pltpu.async_copy(p_hbm.at[sidx.at[0]], msg0, sem0)

    @pl.loop(0, (CPW - 1) // 2)
    def _edges(jj):
        j = jj * 2
        pltpu.async_copy(p_hbm.at[sidx.at[j + 1]], msg1, sem1)
        pltpu.make_async_copy(p_hbm.at[sidx.at[j]], msg0, sem0).wait()
        pltpu.sync_copy(msg0, r_sp.at[didx.at[j]], add=True)
        pltpu.async_copy(p_hbm.at[sidx.at[j + 2]], msg0, sem0)
        pltpu.make_async_copy(p_hbm.at[sidx.at[j + 1]], msg1, sem1).wait()
        pltpu.sync_copy(msg1, r_sp.at[didx.at[j + 1]], add=True)

    pltpu.make_async_copy(p_hbm.at[sidx.at[CPW - 1]], msg0, sem0).wait()
    pltpu.sync_copy(msg0, r_sp.at[didx.at[CPW - 1]], add=True)

    @pl.when(w < 2)
    def _edges_extra():
        pltpu.sync_copy(p_hbm.at[sidx.at[39]], msg1)
        pltpu.sync_copy(msg1, r_sp.at[didx.at[39]], add=True)

    plsc.subcore_barrier()
    for k in range(5):
        pltpu.sync_copy(r_sp.at[pl.ds(s * 625 + k * 125, 125)], bounce)

        @pl.when(c == 0)
        def _out0():
            pltpu.sync_copy(bounce, r0_hbm.at[pl.ds(s * 625 + k * 125, 125)])

        @pl.when(c == 1)
        def _out1():
            pltpu.sync_copy(bounce, r1_hbm.at[pl.ds(s * 625 + k * 125, 125)])


def _t0_body(dp_ref, h0_ref, p_ref, dinv_ref):
    dp = dp_ref[...]
    deg = lax.slice(dp, (0, 0), (N, 1)) + lax.slice(dp, (0, 1), (N, 2)) + 1.0
    dinv = 1.0 / jnp.sqrt(deg)
    dinv_ref[...] = dinv
    p_ref[...] = h0_ref[...] * dinv


_t0 = pl.pallas_call(
    _t0_body,
    out_shape=[
        jax.ShapeDtypeStruct((N, F), jnp.float32),
        jax.ShapeDtypeStruct((N, 1), jnp.float32),
    ],
)


def _gcn_body(r0_ref, r1_ref, p_ref, dinv_ref, w_ref, b_ref, g_ref, be_ref,
              h_ref, pn_ref):
    dinv = dinv_ref[...]
    sm = (r0_ref[...] + r1_ref[...] + p_ref[...]) * dinv
    y = jnp.dot(sm, w_ref[...], preferred_element_type=jnp.float32) + b_ref[...]
    mu = jnp.mean(y, axis=0, keepdims=True)
    d = y - mu
    var = jnp.mean(d * d, axis=0, keepdims=True)
    h = jnp.maximum(d / jnp.sqrt(var + 1e-5) * g_ref[...] + be_ref[...], 0.0)
    h_ref[...] = h
    pn_ref[...] = h * dinv


_gcn = pl.pallas_call(
    _gcn_body,
    out_shape=[
        jax.ShapeDtypeStruct((N, F), jnp.float32),
        jax.ShapeDtypeStruct((N, F), jnp.float32),
    ],
)


def _pool_pairs(y, n_out):
    rows = lax.broadcasted_iota(jnp.int32, (2 * n_out, n_out), 0)
    cols = lax.broadcasted_iota(jnp.int32, (2 * n_out, n_out), 1)
    se = (rows == 2 * cols).astype(jnp.float32)
    so = (rows == 2 * cols + 1).astype(jnp.float32)
    return jnp.maximum(
        jnp.dot(y, se, preferred_element_type=jnp.float32),
        jnp.dot(y, so, preferred_element_type=jnp.float32))


def _head_body(z_ref, w1, b1, w2, b2, w3, b3, out_ref):
    z = z_ref[...]
    y1 = jnp.dot(z, w1[...], preferred_element_type=jnp.float32) + b1[...]
    a = jnp.maximum(_pool_pairs(y1, 400), 0.0)
    y2 = jnp.dot(a, w2[...], preferred_element_type=jnp.float32) + b2[...]
    u = jnp.maximum(_pool_pairs(y2, 50), 0.0)
    out_ref[...] = jnp.dot(u, w3[...],
                           preferred_element_type=jnp.float32) + b3[...]


_head = pl.pallas_call(
    _head_body,
    out_shape=jax.ShapeDtypeStruct((250, 20), jnp.float32),
)


def kernel(x, edge_index, batch, emb, W1, b1, g1, be1, W2, b2, g2, be2,
           L1W, L1b, L2W, L2b, L3W, L3b):
    x_flat = x.reshape(-1)
    src2d = edge_index[0].reshape(NCH, 128)
    dst2d = edge_index[1].reshape(NCH, 128)
    ones_upd = jnp.ones((128, 16), jnp.float32)
    zer_deg = jnp.zeros((640, 16), jnp.float32)
    zer_row = jnp.zeros((125, F), jnp.float32)

    h0_flat, degp = _embdeg(x_flat, emb.reshape(-1), dst2d, ones_upd, zer_deg)
    h0 = h0_flat.reshape(N, F)
    deg_cols = degp[..., 0].reshape(NC, NS * 640)[:, :N].T

    p1, dinv = _t0(deg_cols, h0)

    r1a, r1b = _scatter(p1, src2d, dst2d, zer_row)
    h1, p2 = _gcn(r1a, r1b, p1, dinv, W1, b1.reshape(1, F),
                  g1.reshape(1, F), be1.reshape(1, F))

    r2a, r2b = _scatter(p2, src2d, dst2d, zer_row)
    h2, _ = _gcn(r2a, r2b, p2, dinv, W2, b2.reshape(1, F),
                 g2.reshape(1, F), be2.reshape(1, F))

    z = h2.reshape(250, 40 * F)
    out = _head(z, L1W, L1b.reshape(1, 800), L2W, L2b.reshape(1, 100),
                L3W, L3b.reshape(1, 20))
    return out


# speedup vs baseline: 17.2420x; 1.0152x over previous
import functools

import jax
import jax.numpy as jnp
from jax import lax
from jax.experimental import pallas as pl
from jax.experimental.pallas import tpu as pltpu
from jax.experimental.pallas import tpu_sc as plsc

N = 10000
E = 160000
F = 80
NC = 2
NS = 16
NW = NC * NS

XPT = 12500
XSTG = 12504
NCH = 1250
CPW = 39
DROWS = N + 240

_mesh = plsc.VectorSubcoreMesh(core_axis_name="c", subcore_axis_name="s")
_sc_params = pltpu.CompilerParams(use_tc_tiling_on_sc=False,
                                  needs_layout_passes=False)


@functools.partial(
    pl.kernel,
    out_type=[
        jax.ShapeDtypeStruct((N * F,), jnp.float32),
        jax.ShapeDtypeStruct((NC, NS, 640, 16), jnp.float32),
    ],
    mesh=_mesh,
    scratch_types=[
        pltpu.VMEM((XSTG,), jnp.int32),
        pltpu.VMEM((2 * N,), jnp.float32),
        pltpu.VMEM((2 * XPT,), jnp.float32),
        pltpu.VMEM((40, 128), jnp.int32),
        pltpu.VMEM((128, 16), jnp.float32),
        pltpu.VMEM((640, 16), jnp.float32),
        pltpu.VMEM_SHARED((DROWS, 16), jnp.float32),
        pltpu.SemaphoreType.DMA,
        pltpu.SemaphoreType.DMA,
    ],
    compiler_params=_sc_params,
)
def _embdeg(x_hbm, emb_hbm, dst_hbm, ones_hbm, zer_hbm,
            h0_hbm, degp_hbm, xidx, embv, outb, didx, onesb, bounce, deg_sp,
            deg_sem, emb_sem):
    c = lax.axis_index("c")
    s = lax.axis_index("s")
    w = s * NC + c
    base = 4 * lax.rem(w, 2)

    pltpu.async_copy(emb_hbm, embv, emb_sem)
    xoff = pl.multiple_of(w * XPT - base, 8)
    pltpu.sync_copy(x_hbm.at[pl.ds(xoff, XSTG)], xidx)
    pltpu.sync_copy(dst_hbm.at[pl.ds(w * CPW, CPW)], didx.at[pl.ds(0, CPW)])

    @pl.when(w < 2)
    def _stage_extra():
        pltpu.sync_copy(dst_hbm.at[pl.ds(1248 + w, 1)], didx.at[pl.ds(39, 1)])

    pltpu.sync_copy(ones_hbm, onesb)
    pltpu.sync_copy(zer_hbm, bounce)
    pltpu.sync_copy(bounce, deg_sp.at[pl.ds(s * 640, 640)])
    plsc.subcore_barrier()

    @pl.loop(0, CPW // 3)
    def _deg(jj):
        j = jj * 3
        for t in range(3):
            pltpu.async_copy(onesb, deg_sp.at[didx.at[j + t]], deg_sem, add=True)
        for t in range(3):
            pltpu.make_async_copy(onesb, deg_sp.at[didx.at[j + t]], deg_sem).wait()

    @pl.when(w < 2)
    def _deg_extra():
        pltpu.sync_copy(onesb, deg_sp.at[didx.at[39]], add=True)

    pltpu.make_async_copy(emb_hbm, embv, emb_sem).wait()
    lanes = lax.iota(jnp.int32, 16)

    @pl.loop(0, XPT // 16, unroll=4)
    def _emb(i):
        idx16 = xidx[pl.ds(base + i * 16, 16)]
        fa = idx16 * 2
        a = plsc.load_gather(embv, [fa])
        b = plsc.load_gather(embv, [fa + 1])
        pos = i * 32 + lanes * 2
        plsc.store_scatter(outb, [pos], a)
        plsc.store_scatter(outb, [pos + 1], b)

    tl = xidx[pl.ds(base + XPT - 16, 16)]
    fa = tl * 2
    a = plsc.load_gather(embv, [fa])
    b = plsc.load_gather(embv, [fa + 1])
    pos = 2 * (XPT - 16) + lanes * 2
    plsc.store_scatter(outb, [pos], a)
    plsc.store_scatter(outb, [pos + 1], b)

    pltpu.sync_copy(outb, h0_hbm.at[pl.ds(pl.multiple_of(w * 2 * XPT, 8),
                                          2 * XPT)])
    plsc.subcore_barrier()
    pltpu.sync_copy(deg_sp.at[pl.ds(s * 640, 640)], bounce)
    pltpu.sync_copy(bounce, degp_hbm.at[c].at[s])


@functools.partial(
    pl.kernel,
    out_type=[
        jax.ShapeDtypeStruct((N, F), jnp.float32),
        jax.ShapeDtypeStruct((N, F), jnp.float32),
    ],
    mesh=_mesh,
    scratch_types=[
        pltpu.VMEM((40, 128), jnp.int32),
        pltpu.VMEM((40, 128), jnp.int32),
        pltpu.VMEM((128, F), jnp.float32),
        pltpu.VMEM((128, F), jnp.float32),
        pltpu.VMEM((128, F), jnp.float32),
        pltpu.VMEM((128, F), jnp.float32),
        pltpu.VMEM((125, F), jnp.float32),
        pltpu.VMEM_SHARED((N, F), jnp.float32),
        [pltpu.SemaphoreType.DMA] * 4,
        [pltpu.SemaphoreType.DMA] * 4,
    ],
    compiler_params=_sc_params,
)
def _scatter(p_hbm, src_hbm, dst_hbm, zer_hbm, r0_hbm, r1_hbm,
             sidx, didx, m0, m1, m2, m3, bounce, r_sp, gs, ss):
    c = lax.axis_index("c")
    s = lax.axis_index("s")
    w = s * NC + c
    msg = [m0, m1, m2, m3]

    pltpu.sync_copy(src_hbm.at[pl.ds(w * CPW, CPW)], sidx.at[pl.ds(0, CPW)])
    pltpu.sync_copy(dst_hbm.at[pl.ds(w * CPW, CPW)], didx.at[pl.ds(0, CPW)])

    @pl.when(w < 2)
    def _stage_extra():
        pltpu.sync_copy(src_hbm.at[pl.ds(1248 + w, 1)], sidx.at[pl.ds(39, 1)])
        pltpu.sync_copy(dst_hbm.at[pl.ds(1248 + w, 1)], didx.at[pl.ds(39, 1)])

    pltpu.sync_copy(zer_hbm, bounce)
    for k in range(5):
        pltpu.sync_copy(bounce, r_sp.at[pl.ds(s * 625 + k * 125, 125)])
    plsc.subcore_barrier()

    def gather(j, b):
        pltpu.async_copy(p_hbm.at[sidx.at[j]], msg[b], gs[b])

    def gwait(j, b):
        pltpu.make_async_copy(p_hbm.at[sidx.at[j]], msg[b], gs[b]).wait()

    def scat(j, b):
        pltpu.async_copy(msg[b], r_sp.at[didx.at[j]], ss[b], add=True)

    def swait(j, b):
        pltpu.make_async_copy(msg[b], r_sp.at[didx.at[j]], ss[b]).wait()

    gather(0, 0)
    gather(1, 1)

    @pl.loop(0, 9)
    def _edges(jj):
        j0 = jj * 4
        for b in range(4):
            j = j0 + b
            gwait(j, b)
            scat(j, b)
            b2 = (b + 2) % 4

            @pl.when(j >= 2)
            def _drain():
                swait(j - 2, b2)

            gather(j + 2, b2)

    gwait(36, 0)
    scat(36, 0)
    swait(34, 2)
    gather(38, 2)
    gwait(37, 1)
    scat(37, 1)
    swait(35, 3)

    @pl.when(w < 2)
    def _g39():
        gather(39, 3)

    gwait(38, 2)
    scat(38, 2)
    swait(36, 0)
    swait(37, 1)
    swait(38, 2)

    @pl.when(w < 2)
    def _extra39():
        gwait(39, 3)
        scat(39, 3)
        swait(39, 3)

    plsc.subcore_barrier()
    for k in range(5):
        pltpu.sync_copy(r_sp.at[pl.ds(s * 625 + k * 125, 125)], bounce)

        @pl.when(c == 0)
        def _out0():
            pltpu.sync_copy(bounce, r0_hbm.at[pl.ds(s * 625 + k * 125, 125)])

        @pl.when(c == 1)
        def _out1():
            pltpu.sync_copy(bounce, r1_hbm.at[pl.ds(s * 625 + k * 125, 125)])


def _t0_body(dp_ref, h0_ref, p_ref, dinv_ref):
    dp = dp_ref[...]
    deg = lax.slice(dp, (0, 0), (N, 1)) + lax.slice(dp, (0, 1), (N, 2)) + 1.0
    dinv = 1.0 / jnp.sqrt(deg)
    dinv_ref[...] = dinv
    p_ref[...] = h0_ref[...] * dinv


_t0 = pl.pallas_call(
    _t0_body,
    out_shape=[
        jax.ShapeDtypeStruct((N, F), jnp.float32),
        jax.ShapeDtypeStruct((N, 1), jnp.float32),
    ],
)


def _gcn_body(r0_ref, r1_ref, p_ref, dinv_ref, w_ref, b_ref, g_ref, be_ref,
              h_ref, pn_ref):
    dinv = dinv_ref[...]
    sm = (r0_ref[...] + r1_ref[...] + p_ref[...]) * dinv
    y = jnp.dot(sm, w_ref[...], preferred_element_type=jnp.float32) + b_ref[...]
    mu = jnp.mean(y, axis=0, keepdims=True)
    d = y - mu
    var = jnp.mean(d * d, axis=0, keepdims=True)
    h = jnp.maximum(d / jnp.sqrt(var + 1e-5) * g_ref[...] + be_ref[...], 0.0)
    h_ref[...] = h
    pn_ref[...] = h * dinv


_gcn = pl.pallas_call(
    _gcn_body,
    out_shape=[
        jax.ShapeDtypeStruct((N, F), jnp.float32),
        jax.ShapeDtypeStruct((N, F), jnp.float32),
    ],
)


def _pool_pairs(y, n_out):
    rows = lax.broadcasted_iota(jnp.int32, (2 * n_out, n_out), 0)
    cols = lax.broadcasted_iota(jnp.int32, (2 * n_out, n_out), 1)
    se = (rows == 2 * cols).astype(jnp.float32)
    so = (rows == 2 * cols + 1).astype(jnp.float32)
    return jnp.maximum(
        jnp.dot(y, se, preferred_element_type=jnp.float32),
        jnp.dot(y, so, preferred_element_type=jnp.float32))


def _head_body(z_ref, w1, b1, w2, b2, w3, b3, out_ref):
    z = z_ref[...]
    y1 = jnp.dot(z, w1[...], preferred_element_type=jnp.float32) + b1[...]
    a = jnp.maximum(_pool_pairs(y1, 400), 0.0)
    y2 = jnp.dot(a, w2[...], preferred_element_type=jnp.float32) + b2[...]
    u = jnp.maximum(_pool_pairs(y2, 50), 0.0)
    out_ref[...] = jnp.dot(u, w3[...],
                           preferred_element_type=jnp.float32) + b3[...]


_head = pl.pallas_call(
    _head_body,
    out_shape=jax.ShapeDtypeStruct((250, 20), jnp.float32),
)


def kernel(x, edge_index, batch, emb, W1, b1, g1, be1, W2, b2, g2, be2,
           L1W, L1b, L2W, L2b, L3W, L3b):
    x_flat = x.reshape(-1)
    src2d = edge_index[0].reshape(NCH, 128)
    dst2d = edge_index[1].reshape(NCH, 128)
    ones_upd = jnp.ones((128, 16), jnp.float32)
    zer_deg = jnp.zeros((640, 16), jnp.float32)
    zer_row = jnp.zeros((125, F), jnp.float32)

    h0_flat, degp = _embdeg(x_flat, emb.reshape(-1), dst2d, ones_upd, zer_deg)
    h0 = h0_flat.reshape(N, F)
    deg_cols = degp[..., 0].reshape(NC, NS * 640)[:, :N].T

    p1, dinv = _t0(deg_cols, h0)

    r1a, r1b = _scatter(p1, src2d, dst2d, zer_row)
    h1, p2 = _gcn(r1a, r1b, p1, dinv, W1, b1.reshape(1, F),
                  g1.reshape(1, F), be1.reshape(1, F))

    r2a, r2b = _scatter(p2, src2d, dst2d, zer_row)
    h2, _ = _gcn(r2a, r2b, p2, dinv, W2, b2.reshape(1, F),
                 g2.reshape(1, F), be2.reshape(1, F))

    z = h2.reshape(250, 40 * F)
    out = _head(z, L1W, L1b.reshape(1, 800), L2W, L2b.reshape(1, 100),
                L3W, L3b.reshape(1, 20))
    return out
